# baseline (speedup 1.0000x reference)
import functools

import jax
import jax.numpy as jnp
from jax import lax
from jax.experimental import pallas as pl
from jax.experimental.pallas import tpu as pltpu
from jax.experimental.pallas import tpu_sc as plsc

_B = 4096
_EMB = 256

_info = plsc.get_sparse_core_info()
_NC = _info.num_cores
_NS = _info.num_subcores
_NW = _NC * _NS
_B_PER_W = _B // _NW


@functools.partial(
    pl.kernel,
    mesh=plsc.VectorSubcoreMesh(core_axis_name="c", subcore_axis_name="s"),
    out_type=jax.ShapeDtypeStruct((_B, 1, 2, _EMB), jnp.float32),
    scratch_types=[
        pltpu.VMEM((2, _B_PER_W), jnp.int32),
        pltpu.VMEM((_B_PER_W, _EMB), jnp.float32),
        pltpu.VMEM((_B_PER_W, _EMB), jnp.float32),
        pltpu.SemaphoreType.DMA,
        pltpu.SemaphoreType.DMA,
    ],
)
def _embed(x_hbm, table_hbm, out_hbm, idx_v, rows1_v, rows2_v, sem_g, sem_s):
    wid = lax.axis_index("s") * _NC + lax.axis_index("c")
    base = wid * _B_PER_W
    pltpu.sync_copy(x_hbm.at[:, pl.ds(base, _B_PER_W)], idx_v)
    half = _B_PER_W // 2
    rows = (rows1_v, rows2_v)
    gathers = []
    for p in (0, 1):
        for h in (0, 1):
            gathers.append((p, h, pltpu.async_copy(
                table_hbm.at[idx_v.at[p, pl.ds(h * half, half)]],
                rows[p].at[pl.ds(h * half, half)],
                sem_g,
            )))
    stores = []
    for p, h, g in gathers:
        g.wait()
        stores.append(pltpu.async_copy(
            rows[p].at[pl.ds(h * half, half)],
            out_hbm.at[pl.ds(base + h * half, half), 0, p, :],
            sem_s,
        ))
    for s in stores:
        s.wait()


def kernel(x, table):
    return _embed(x, table)


# speedup vs baseline: 1.0143x; 1.0143x over previous
import functools

import jax
import jax.numpy as jnp
from jax import lax
from jax.experimental import pallas as pl
from jax.experimental.pallas import tpu as pltpu
from jax.experimental.pallas import tpu_sc as plsc

_B = 4096
_EMB = 256

_info = plsc.get_sparse_core_info()
_NC = _info.num_cores
_NS = _info.num_subcores
_NW = _NC * _NS
_B_PER_W = _B // _NW


@functools.partial(
    pl.kernel,
    mesh=plsc.VectorSubcoreMesh(core_axis_name="c", subcore_axis_name="s"),
    out_type=jax.ShapeDtypeStruct((_B, 1, 2, _EMB), jnp.float32),
    scratch_types=[
        pltpu.VMEM((2, _B_PER_W), jnp.int32),
        pltpu.VMEM((_B_PER_W, _EMB), jnp.float32),
        pltpu.VMEM((_B_PER_W, _EMB), jnp.float32),
        pltpu.SemaphoreType.DMA,
        pltpu.SemaphoreType.DMA,
    ],
)
def _embed(x_hbm, table_hbm, out_hbm, idx_v, rows1_v, rows2_v, sem_g, sem_s):
    wid = lax.axis_index("s") * _NC + lax.axis_index("c")
    base = wid * _B_PER_W
    pltpu.sync_copy(x_hbm.at[:, pl.ds(base, _B_PER_W)], idx_v)
    g1 = pltpu.async_copy(table_hbm.at[idx_v.at[0]], rows1_v, sem_g)
    g2 = pltpu.async_copy(table_hbm.at[idx_v.at[1]], rows2_v, sem_g)
    g1.wait()
    g2.wait()
    s1 = pltpu.async_copy(
        rows1_v, out_hbm.at[pl.ds(base, _B_PER_W), 0, 0, :], sem_s)
    s2 = pltpu.async_copy(
        rows2_v, out_hbm.at[pl.ds(base, _B_PER_W), 0, 1, :], sem_s)
    s1.wait()
    s2.wait()


def kernel(x, table):
    return _embed(x, table)
